# baseline (speedup 1.0000x reference)
import functools

import jax
import jax.numpy as jnp
from jax import lax
from jax.experimental import pallas as pl
from jax.experimental.pallas import tpu as pltpu
from jax.experimental.pallas import tpu_sc as plsc

NC = 2
NS = 16
LW = 16
CHUNK = 128
RCL = 512
TB = 64


def _tc_matmul1(x, W1, n_pad):
    RP, EMB, HID = W1.shape
    N = x.shape[0]

    def body(x_ref, w_ref, y8_ref, ys_ref):
        m = jnp.dot(x_ref[...], w_ref[0], preferred_element_type=jnp.float32)
        mp = jnp.pad(m, ((0, n_pad - N), (0, 0)))
        st = n_pad // 8
        y8_ref[...] = jnp.concatenate(
            [mp[k * st:(k + 1) * st] for k in range(8)], axis=1)
        ys_ref[...] = m

    return pl.pallas_call(
        body,
        grid=(RP,),
        in_specs=[
            pl.BlockSpec((N, EMB), lambda r: (0, 0)),
            pl.BlockSpec((1, EMB, HID), lambda r: (r, 0, 0)),
        ],
        out_specs=[
            pl.BlockSpec((n_pad // 8, 8 * HID), lambda r: (r, 0)),
            pl.BlockSpec((N, HID), lambda r: (0, 0)),
        ],
        out_shape=[
            jax.ShapeDtypeStruct((RP * n_pad // 8, 8 * HID), jnp.float32),
            jax.ShapeDtypeStruct((N, HID), jnp.float32),
        ],
    )(x, W1)


def _tc_norm_matmul2(acc, yself, b1, g1, bb1, W2p, n, n_pad):
    RP, HID, CP = W2p.shape
    NP = acc.shape[1]

    def body(a_ref, y_ref, b_ref, g_ref, bb_ref, w_ref, z8_ref, zs_ref,
             h_ref):
        @pl.when(pl.program_id(0) == 0)
        def _():
            a = (a_ref[0] + a_ref[1])[:n] + y_ref[...] + b_ref[0]
            mu = jnp.mean(a, axis=-1, keepdims=True)
            var = jnp.mean((a - mu) ** 2, axis=-1, keepdims=True)
            h = (a - mu) * lax.rsqrt(var + 1e-5) * g_ref[0] + bb_ref[0]
            h_ref[...] = jnp.maximum(h, 0.0)

        zr = jnp.dot(h_ref[...], w_ref[0], preferred_element_type=jnp.float32)
        zp = jnp.pad(zr, ((0, n_pad - n), (0, 0)))
        st = n_pad // 8
        z8_ref[...] = jnp.concatenate(
            [zp[k * st:(k + 1) * st] for k in range(8)], axis=1)
        zs_ref[...] = zr

    return pl.pallas_call(
        body,
        grid=(RP,),
        in_specs=[
            pl.BlockSpec((2, NP, HID), lambda r: (0, 0, 0)),
            pl.BlockSpec((n, HID), lambda r: (0, 0)),
            pl.BlockSpec((1, HID), lambda r: (0, 0)),
            pl.BlockSpec((1, HID), lambda r: (0, 0)),
            pl.BlockSpec((1, HID), lambda r: (0, 0)),
            pl.BlockSpec((1, HID, CP), lambda r: (r, 0, 0)),
        ],
        out_specs=[
            pl.BlockSpec((n_pad // 8, 8 * CP), lambda r: (r, 0)),
            pl.BlockSpec((n, CP), lambda r: (0, 0)),
        ],
        out_shape=[
            jax.ShapeDtypeStruct((RP * n_pad // 8, 8 * CP), jnp.float32),
            jax.ShapeDtypeStruct((n, CP), jnp.float32),
        ],
        scratch_shapes=[pltpu.VMEM((n, HID), jnp.float32)],
    )(acc, yself, b1, g1, bb1, W2p)


def _tc_final_norm(acc, zself, b2, g2, bb2, n, ncls):
    NP, CP = acc.shape[1], acc.shape[2]

    def body(a_ref, z_ref, b_ref, g_ref, bb_ref, o_ref):
        a = ((a_ref[0] + a_ref[1])[:n] + z_ref[...])[:, :ncls] + b_ref[0]
        mu = jnp.mean(a, axis=-1, keepdims=True)
        var = jnp.mean((a - mu) ** 2, axis=-1, keepdims=True)
        o_ref[...] = (a - mu) * lax.rsqrt(var + 1e-5) * g_ref[0] + bb_ref[0]

    return pl.pallas_call(
        body,
        in_specs=[
            pl.BlockSpec((2, NP, CP), lambda: (0, 0, 0)),
            pl.BlockSpec((n, CP), lambda: (0, 0)),
            pl.BlockSpec((1, ncls), lambda: (0, 0)),
            pl.BlockSpec((1, ncls), lambda: (0, 0)),
            pl.BlockSpec((1, ncls), lambda: (0, 0)),
        ],
        out_specs=pl.BlockSpec((n, ncls), lambda: (0, 0)),
        out_shape=jax.ShapeDtypeStruct((n, ncls), jnp.float32),
    )(acc, zself, b2, g2, bb2)


def _make_edge_pass(n_pad, n_table, nt_stage, nch):
    rows_per_sub = n_pad // NS
    rc = 128
    n_rc = rows_per_sub // rc
    nt_sub = nt_stage // NS
    n_lc = nt_sub // RCL
    mesh = plsc.VectorSubcoreMesh(core_axis_name="c", subcore_axis_name="s")

    @functools.partial(
        pl.kernel,
        mesh=mesh,
        compiler_params=pltpu.CompilerParams(use_tc_tiling_on_sc=False),
        out_type=jax.ShapeDtypeStruct((NC, n_pad, LW), jnp.float32),
        scratch_types=[
            pltpu.VMEM((nch, CHUNK), jnp.int32),
            pltpu.VMEM((nch, CHUNK), jnp.float32),
            pltpu.VMEM((1, CHUNK), jnp.int32),
            pltpu.VMEM((1, CHUNK), jnp.int32),
            pltpu.VMEM((CHUNK, LW), jnp.float32),
            pltpu.VMEM((TB, 8 * LW), jnp.float32),
            pltpu.VMEM_SHARED((nt_stage, LW), jnp.float32),
            pltpu.VMEM_SHARED((n_pad, LW), jnp.float32),
            pltpu.SemaphoreType.DMA,
        ],
    )
    def edge_pass(table, combo, vals, out,
                  combo_v, vals_v, gidx_c, src_c, rb0, buf8, tab_sh, acc,
                  sm0):
        c = lax.axis_index("c")
        s = lax.axis_index("s")

        pltpu.sync_copy(combo.at[c, s], combo_v)
        pltpu.sync_copy(vals.at[c, s], vals_v)

        st = n_pad // 8
        rel_per_core = nt_stage // n_pad
        spr = n_pad // nt_sub
        tpt = nt_sub // 8
        rl = s // spr
        for i in range(tpt // TB):
            g0 = (s % spr) * tpt + i * TB
            pltpu.sync_copy(
                table.at[pl.ds((c * rel_per_core + rl) * st + g0, TB)], buf8)
            for k in range(8):
                pltpu.sync_copy(
                    buf8.at[pl.ds(0, TB), pl.ds(k * LW, LW)],
                    tab_sh.at[pl.ds(rl * n_pad + k * st + g0, TB)])

        def zrow(i, carry):
            rb0[i, :] = jnp.zeros((LW,), jnp.float32)
            return carry
        lax.fori_loop(0, rc, zrow, 0)
        for t in range(n_rc):
            pltpu.sync_copy(rb0,
                            acc.at[pl.ds(s * rows_per_sub + t * rc, rc)])
        plsc.subcore_barrier()

        def chunk_body(j, carry):
            for g in range(CHUNK // LW):
                v = combo_v[j, pl.ds(g * LW, LW)]
                gidx_c[0, pl.ds(g * LW, LW)] = lax.shift_right_logical(v, 14)
                src_c[0, pl.ds(g * LW, LW)] = lax.bitwise_and(v, 16383)
            pltpu.async_copy(tab_sh.at[gidx_c.at[0]], rb0, sm0).wait()
            for g in range(CHUNK // LW):
                v16 = vals_v[j, pl.ds(g * LW, LW)]
                for k in range(LW):
                    r = g * LW + k
                    bc = jnp.full((LW,), v16[k], jnp.float32)
                    rb0[r, :] = rb0[r, :] * bc
            pltpu.sync_copy(rb0, acc.at[src_c.at[0]], add=True)
            return carry
        lax.fori_loop(0, nch, chunk_body, 0)
        plsc.subcore_barrier()

        for t in range(n_rc):
            base = s * rows_per_sub + t * rc
            pltpu.sync_copy(acc.at[pl.ds(base, rc)], rb0)
            pltpu.sync_copy(rb0, out.at[c, pl.ds(base, rc)])

    return edge_pass


def kernel(features, W1, W2, bias1, bias2, ln1_g, ln1_b, ln2_g, ln2_b,
           rows, cols, vals):
    N, EMB = features.shape
    RP, _, HID = W1.shape
    NCLS = W2.shape[2]
    E = rows.shape[0]
    R = (RP - 1) // 2
    T = (E - N) // 2

    n_pad = -(-N // (NS * 128)) * NS * 128
    rows32 = rows.astype(jnp.int32)
    cols32 = cols.astype(jnp.int32)
    src = rows32 % N
    gidx = (rows32 - src) // N * n_pad + cols32

    n2r = 2 * R * n_pad
    nt_stage = R * n_pad
    rebase = n2r - nt_stage
    nch = -(-T // (NS * CHUNK))
    epc = NS * nch * CHUNK

    def part(a0, a1):
        a = jnp.concatenate([
            jnp.pad(a0, (0, epc - T)), jnp.pad(a1, (0, epc - T))])
        return a.reshape(NC, NS, nch, CHUNK)

    combo = gidx * 16384 + src
    combo4 = part(combo[:T], combo[T:2 * T] - rebase * 16384)
    vals32 = vals.astype(jnp.float32)
    vals4 = part(vals32[:T], vals32[T:2 * T])

    edge_pass = _make_edge_pass(n_pad, n2r, nt_stage, nch)

    y8, ys = _tc_matmul1(features.astype(jnp.float32), W1, n_pad)
    acc1 = edge_pass(y8, combo4, vals4)

    W2p = jnp.pad(W2, ((0, 0), (0, 0), (0, LW - NCLS)))
    z8, zs = _tc_norm_matmul2(acc1, ys, bias1.reshape(1, HID),
                              ln1_g.reshape(1, HID), ln1_b.reshape(1, HID),
                              W2p, N, n_pad)
    acc2 = edge_pass(z8, combo4, vals4)

    return _tc_final_norm(acc2, zs, bias2.reshape(1, NCLS),
                          ln2_g.reshape(1, NCLS), ln2_b.reshape(1, NCLS),
                          N, NCLS)


# speedup vs baseline: 1.0838x; 1.0838x over previous
import functools

import jax
import jax.numpy as jnp
from jax import lax
from jax.experimental import pallas as pl
from jax.experimental.pallas import tpu as pltpu
from jax.experimental.pallas import tpu_sc as plsc

NC = 2
NS = 16
LW = 16
CHUNK = 128
RCL = 512
TB = 64


def _tc_matmul1(x, W1, n_pad):
    RP, EMB, HID = W1.shape
    N = x.shape[0]

    def body(x_ref, w_ref, y8_ref, ys_ref):
        m = jnp.dot(x_ref[...], w_ref[0], preferred_element_type=jnp.float32)
        mp = jnp.pad(m, ((0, n_pad - N), (0, 0)))
        st = n_pad // 8
        y8_ref[...] = jnp.concatenate(
            [mp[k * st:(k + 1) * st] for k in range(8)], axis=1)
        ys_ref[...] = m

    return pl.pallas_call(
        body,
        grid=(RP,),
        in_specs=[
            pl.BlockSpec((N, EMB), lambda r: (0, 0)),
            pl.BlockSpec((1, EMB, HID), lambda r: (r, 0, 0)),
        ],
        out_specs=[
            pl.BlockSpec((n_pad // 8, 8 * HID), lambda r: (r, 0)),
            pl.BlockSpec((N, HID), lambda r: (0, 0)),
        ],
        out_shape=[
            jax.ShapeDtypeStruct((RP * n_pad // 8, 8 * HID), jnp.float32),
            jax.ShapeDtypeStruct((N, HID), jnp.float32),
        ],
    )(x, W1)


def _tc_norm_matmul2(acc, yself, b1, g1, bb1, W2p, n, n_pad):
    RP, HID, CP = W2p.shape
    NP = acc.shape[1]

    def body(a_ref, y_ref, b_ref, g_ref, bb_ref, w_ref, z8_ref, zs_ref,
             h_ref):
        @pl.when(pl.program_id(0) == 0)
        def _():
            a = (a_ref[0] + a_ref[1])[:n] + y_ref[...] + b_ref[0]
            mu = jnp.mean(a, axis=-1, keepdims=True)
            var = jnp.mean((a - mu) ** 2, axis=-1, keepdims=True)
            h = (a - mu) * lax.rsqrt(var + 1e-5) * g_ref[0] + bb_ref[0]
            h_ref[...] = jnp.maximum(h, 0.0)

        zr = jnp.dot(h_ref[...], w_ref[0], preferred_element_type=jnp.float32)
        zp = jnp.pad(zr, ((0, n_pad - n), (0, 0)))
        st = n_pad // 8
        z8_ref[...] = jnp.concatenate(
            [zp[k * st:(k + 1) * st] for k in range(8)], axis=1)
        zs_ref[...] = zr

    return pl.pallas_call(
        body,
        grid=(RP,),
        in_specs=[
            pl.BlockSpec((2, NP, HID), lambda r: (0, 0, 0)),
            pl.BlockSpec((n, HID), lambda r: (0, 0)),
            pl.BlockSpec((1, HID), lambda r: (0, 0)),
            pl.BlockSpec((1, HID), lambda r: (0, 0)),
            pl.BlockSpec((1, HID), lambda r: (0, 0)),
            pl.BlockSpec((1, HID, CP), lambda r: (r, 0, 0)),
        ],
        out_specs=[
            pl.BlockSpec((n_pad // 8, 8 * CP), lambda r: (r, 0)),
            pl.BlockSpec((n, CP), lambda r: (0, 0)),
        ],
        out_shape=[
            jax.ShapeDtypeStruct((RP * n_pad // 8, 8 * CP), jnp.float32),
            jax.ShapeDtypeStruct((n, CP), jnp.float32),
        ],
        scratch_shapes=[pltpu.VMEM((n, HID), jnp.float32)],
    )(acc, yself, b1, g1, bb1, W2p)


def _tc_final_norm(acc, zself, b2, g2, bb2, n, ncls):
    NP, CP = acc.shape[1], acc.shape[2]

    def body(a_ref, z_ref, b_ref, g_ref, bb_ref, o_ref):
        a = ((a_ref[0] + a_ref[1])[:n] + z_ref[...])[:, :ncls] + b_ref[0]
        mu = jnp.mean(a, axis=-1, keepdims=True)
        var = jnp.mean((a - mu) ** 2, axis=-1, keepdims=True)
        o_ref[...] = (a - mu) * lax.rsqrt(var + 1e-5) * g_ref[0] + bb_ref[0]

    return pl.pallas_call(
        body,
        in_specs=[
            pl.BlockSpec((2, NP, CP), lambda: (0, 0, 0)),
            pl.BlockSpec((n, CP), lambda: (0, 0)),
            pl.BlockSpec((1, ncls), lambda: (0, 0)),
            pl.BlockSpec((1, ncls), lambda: (0, 0)),
            pl.BlockSpec((1, ncls), lambda: (0, 0)),
        ],
        out_specs=pl.BlockSpec((n, ncls), lambda: (0, 0)),
        out_shape=jax.ShapeDtypeStruct((n, ncls), jnp.float32),
    )(acc, zself, b2, g2, bb2)


def _make_edge_pass(n_pad, n_table, nt_stage, nch):
    rows_per_sub = n_pad // NS
    rc = 128
    n_rc = rows_per_sub // rc
    nt_sub = nt_stage // NS
    n_lc = nt_sub // RCL
    mesh = plsc.VectorSubcoreMesh(core_axis_name="c", subcore_axis_name="s")

    @functools.partial(
        pl.kernel,
        mesh=mesh,
        compiler_params=pltpu.CompilerParams(use_tc_tiling_on_sc=False),
        out_type=jax.ShapeDtypeStruct((NC, n_pad, LW), jnp.float32),
        scratch_types=[
            pltpu.VMEM((nch, CHUNK), jnp.int32),
            pltpu.VMEM((nch, CHUNK), jnp.float32),
            pltpu.VMEM((1, CHUNK), jnp.int32),
            pltpu.VMEM((1, CHUNK), jnp.int32),
            pltpu.VMEM((1, CHUNK), jnp.int32),
            pltpu.VMEM((1, CHUNK), jnp.int32),
            pltpu.VMEM((CHUNK, LW), jnp.float32),
            pltpu.VMEM((CHUNK, LW), jnp.float32),
            pltpu.VMEM((TB, 8 * LW), jnp.float32),
            pltpu.VMEM_SHARED((nt_stage, LW), jnp.float32),
            pltpu.VMEM_SHARED((n_pad, LW), jnp.float32),
            pltpu.SemaphoreType.DMA,
            pltpu.SemaphoreType.DMA,
            pltpu.SemaphoreType.DMA,
        ],
    )
    def edge_pass(table, combo, vals, out,
                  combo_v, vals_v, gidx_c, src_c, gidx_c2, src_c2, rb0, rb1,
                  buf8, tab_sh, acc, sm0, ss0, ss1):
        c = lax.axis_index("c")
        s = lax.axis_index("s")

        pltpu.sync_copy(combo.at[c, s], combo_v)
        pltpu.sync_copy(vals.at[c, s], vals_v)

        st = n_pad // 8
        rel_per_core = nt_stage // n_pad
        spr = n_pad // nt_sub
        tpt = nt_sub // 8
        rl = s // spr
        for i in range(tpt // TB):
            g0 = (s % spr) * tpt + i * TB
            pltpu.sync_copy(
                table.at[pl.ds((c * rel_per_core + rl) * st + g0, TB)], buf8)
            hs = [pltpu.async_copy(
                      buf8.at[pl.ds(0, TB), pl.ds(k * LW, LW)],
                      tab_sh.at[pl.ds(rl * n_pad + k * st + g0, TB)], sm0)
                  for k in range(8)]
            for h in hs:
                h.wait()

        def zrow(i, carry):
            rb0[i, :] = jnp.zeros((LW,), jnp.float32)
            return carry
        lax.fori_loop(0, rc, zrow, 0)
        for t in range(n_rc):
            pltpu.sync_copy(rb0,
                            acc.at[pl.ds(s * rows_per_sub + t * rc, rc)])
        plsc.subcore_barrier()

        slots = ((rb0, gidx_c, src_c, ss0), (rb1, gidx_c2, src_c2, ss1))

        def chunk_half(q, j, rb, gc, sc, ss):
            @pl.when(q > 0)
            def _():
                pltpu.make_async_copy(rb, acc.at[sc.at[0]], ss).wait()
            for g in range(CHUNK // LW):
                v = combo_v[j, pl.ds(g * LW, LW)]
                gc[0, pl.ds(g * LW, LW)] = lax.shift_right_logical(v, 14)
                sc[0, pl.ds(g * LW, LW)] = lax.bitwise_and(v, 16383)
            pltpu.async_copy(tab_sh.at[gc.at[0]], rb, sm0).wait()
            for g in range(CHUNK // LW):
                v16 = vals_v[j, pl.ds(g * LW, LW)]
                for k in range(LW):
                    r = g * LW + k
                    bc = jnp.full((LW,), v16[k], jnp.float32)
                    rb[r, :] = rb[r, :] * bc
            pltpu.async_copy(rb, acc.at[sc.at[0]], ss, add=True)

        def chunk_body(q, carry):
            for b, (rb, gc, sc, ss) in enumerate(slots):
                chunk_half(q, 2 * q + b, rb, gc, sc, ss)
            return carry
        lax.fori_loop(0, nch // 2, chunk_body, 0)
        for rb, gc, sc, ss in slots:
            pltpu.make_async_copy(rb, acc.at[sc.at[0]], ss).wait()
        plsc.subcore_barrier()

        for t in range(n_rc):
            base = s * rows_per_sub + t * rc
            pltpu.sync_copy(acc.at[pl.ds(base, rc)], rb0)
            pltpu.sync_copy(rb0, out.at[c, pl.ds(base, rc)])

    return edge_pass


def kernel(features, W1, W2, bias1, bias2, ln1_g, ln1_b, ln2_g, ln2_b,
           rows, cols, vals):
    N, EMB = features.shape
    RP, _, HID = W1.shape
    NCLS = W2.shape[2]
    E = rows.shape[0]
    R = (RP - 1) // 2
    T = (E - N) // 2

    n_pad = -(-N // (NS * 128)) * NS * 128
    rows32 = rows.astype(jnp.int32)
    cols32 = cols.astype(jnp.int32)
    src = rows32 % N
    gidx = (rows32 - src) // N * n_pad + cols32

    n2r = 2 * R * n_pad
    nt_stage = R * n_pad
    rebase = n2r - nt_stage
    nch = -(-(-(-T // (NS * CHUNK))) // 2) * 2
    epc = NS * nch * CHUNK

    def part(a0, a1):
        a = jnp.concatenate([
            jnp.pad(a0, (0, epc - T)), jnp.pad(a1, (0, epc - T))])
        return a.reshape(NC, NS, nch, CHUNK)

    combo = gidx * 16384 + src
    combo4 = part(combo[:T], combo[T:2 * T] - rebase * 16384)
    vals32 = vals.astype(jnp.float32)
    vals4 = part(vals32[:T], vals32[T:2 * T])

    edge_pass = _make_edge_pass(n_pad, n2r, nt_stage, nch)

    y8, ys = _tc_matmul1(features.astype(jnp.float32), W1, n_pad)
    acc1 = edge_pass(y8, combo4, vals4)

    W2p = jnp.pad(W2, ((0, 0), (0, 0), (0, LW - NCLS)))
    z8, zs = _tc_norm_matmul2(acc1, ys, bias1.reshape(1, HID),
                              ln1_g.reshape(1, HID), ln1_b.reshape(1, HID),
                              W2p, N, n_pad)
    acc2 = edge_pass(z8, combo4, vals4)

    return _tc_final_norm(acc2, zs, bias2.reshape(1, NCLS),
                          ln2_g.reshape(1, NCLS), ln2_b.reshape(1, NCLS),
                          N, NCLS)


# speedup vs baseline: 1.0851x; 1.0012x over previous
import functools

import jax
import jax.numpy as jnp
from jax import lax
from jax.experimental import pallas as pl
from jax.experimental.pallas import tpu as pltpu
from jax.experimental.pallas import tpu_sc as plsc

NC = 2
NS = 16
LW = 16
CHUNK = 128
RCL = 512
TB = 64


def _tc_matmul1(x, W1, n_pad):
    RP, EMB, HID = W1.shape
    N = x.shape[0]

    def body(x_ref, w_ref, y8_ref, ys_ref):
        st = n_pad // 8
        for r in range(RP):
            m = jnp.dot(x_ref[...], w_ref[r],
                        preferred_element_type=jnp.float32)
            mp = jnp.pad(m, ((0, n_pad - N), (0, 0)))
            y8_ref[pl.ds(r * st, st), :] = jnp.concatenate(
                [mp[k * st:(k + 1) * st] for k in range(8)], axis=1)
            if r == RP - 1:
                ys_ref[...] = m

    return pl.pallas_call(
        body,
        in_specs=[
            pl.BlockSpec((N, EMB), lambda: (0, 0)),
            pl.BlockSpec((RP, EMB, HID), lambda: (0, 0, 0)),
        ],
        out_specs=[
            pl.BlockSpec((RP * n_pad // 8, 8 * HID), lambda: (0, 0)),
            pl.BlockSpec((N, HID), lambda: (0, 0)),
        ],
        out_shape=[
            jax.ShapeDtypeStruct((RP * n_pad // 8, 8 * HID), jnp.float32),
            jax.ShapeDtypeStruct((N, HID), jnp.float32),
        ],
        compiler_params=pltpu.CompilerParams(
            vmem_limit_bytes=100 * 1024 * 1024),
    )(x, W1)


def _tc_norm_matmul2(acc, yself, b1, g1, bb1, W2p, n, n_pad):
    RP, HID, CP = W2p.shape
    NP = acc.shape[1]

    def body(a_ref, y_ref, b_ref, g_ref, bb_ref, w_ref, z8_ref, zs_ref,
             h_ref):
        @pl.when(pl.program_id(0) == 0)
        def _():
            a = (a_ref[0] + a_ref[1])[:n] + y_ref[...] + b_ref[0]
            mu = jnp.mean(a, axis=-1, keepdims=True)
            var = jnp.mean((a - mu) ** 2, axis=-1, keepdims=True)
            h = (a - mu) * lax.rsqrt(var + 1e-5) * g_ref[0] + bb_ref[0]
            h_ref[...] = jnp.maximum(h, 0.0)

        zr = jnp.dot(h_ref[...], w_ref[0], preferred_element_type=jnp.float32)
        zp = jnp.pad(zr, ((0, n_pad - n), (0, 0)))
        st = n_pad // 8
        z8_ref[...] = jnp.concatenate(
            [zp[k * st:(k + 1) * st] for k in range(8)], axis=1)
        zs_ref[...] = zr

    return pl.pallas_call(
        body,
        grid=(RP,),
        in_specs=[
            pl.BlockSpec((2, NP, HID), lambda r: (0, 0, 0)),
            pl.BlockSpec((n, HID), lambda r: (0, 0)),
            pl.BlockSpec((1, HID), lambda r: (0, 0)),
            pl.BlockSpec((1, HID), lambda r: (0, 0)),
            pl.BlockSpec((1, HID), lambda r: (0, 0)),
            pl.BlockSpec((1, HID, CP), lambda r: (r, 0, 0)),
        ],
        out_specs=[
            pl.BlockSpec((n_pad // 8, 8 * CP), lambda r: (r, 0)),
            pl.BlockSpec((n, CP), lambda r: (0, 0)),
        ],
        out_shape=[
            jax.ShapeDtypeStruct((RP * n_pad // 8, 8 * CP), jnp.float32),
            jax.ShapeDtypeStruct((n, CP), jnp.float32),
        ],
        scratch_shapes=[pltpu.VMEM((n, HID), jnp.float32)],
    )(acc, yself, b1, g1, bb1, W2p)


def _tc_final_norm(acc, zself, b2, g2, bb2, n, ncls):
    NP, CP = acc.shape[1], acc.shape[2]

    def body(a_ref, z_ref, b_ref, g_ref, bb_ref, o_ref):
        a = ((a_ref[0] + a_ref[1])[:n] + z_ref[...])[:, :ncls] + b_ref[0]
        mu = jnp.mean(a, axis=-1, keepdims=True)
        var = jnp.mean((a - mu) ** 2, axis=-1, keepdims=True)
        o_ref[...] = (a - mu) * lax.rsqrt(var + 1e-5) * g_ref[0] + bb_ref[0]

    return pl.pallas_call(
        body,
        in_specs=[
            pl.BlockSpec((2, NP, CP), lambda: (0, 0, 0)),
            pl.BlockSpec((n, CP), lambda: (0, 0)),
            pl.BlockSpec((1, ncls), lambda: (0, 0)),
            pl.BlockSpec((1, ncls), lambda: (0, 0)),
            pl.BlockSpec((1, ncls), lambda: (0, 0)),
        ],
        out_specs=pl.BlockSpec((n, ncls), lambda: (0, 0)),
        out_shape=jax.ShapeDtypeStruct((n, ncls), jnp.float32),
    )(acc, zself, b2, g2, bb2)


def _make_edge_pass(n_pad, n_table, nt_stage, nch):
    rows_per_sub = n_pad // NS
    rc = 128
    n_rc = rows_per_sub // rc
    nt_sub = nt_stage // NS
    n_lc = nt_sub // RCL
    mesh = plsc.VectorSubcoreMesh(core_axis_name="c", subcore_axis_name="s")

    @functools.partial(
        pl.kernel,
        mesh=mesh,
        compiler_params=pltpu.CompilerParams(use_tc_tiling_on_sc=False),
        out_type=jax.ShapeDtypeStruct((NC, n_pad, LW), jnp.float32),
        scratch_types=[
            pltpu.VMEM((nch, CHUNK), jnp.int32),
            pltpu.VMEM((nch, CHUNK), jnp.float32),
            pltpu.VMEM((1, CHUNK), jnp.int32),
            pltpu.VMEM((1, CHUNK), jnp.int32),
            pltpu.VMEM((1, CHUNK), jnp.int32),
            pltpu.VMEM((1, CHUNK), jnp.int32),
            pltpu.VMEM((CHUNK, LW), jnp.float32),
            pltpu.VMEM((CHUNK, LW), jnp.float32),
            pltpu.VMEM((TB, 8 * LW), jnp.float32),
            pltpu.VMEM_SHARED((nt_stage, LW), jnp.float32),
            pltpu.VMEM_SHARED((n_pad, LW), jnp.float32),
            pltpu.SemaphoreType.DMA,
            pltpu.SemaphoreType.DMA,
            pltpu.SemaphoreType.DMA,
        ],
    )
    def edge_pass(table, combo, vals, out,
                  combo_v, vals_v, gidx_c, src_c, gidx_c2, src_c2, rb0, rb1,
                  buf8, tab_sh, acc, sm0, ss0, ss1):
        c = lax.axis_index("c")
        s = lax.axis_index("s")

        pltpu.sync_copy(combo.at[c, s], combo_v)
        pltpu.sync_copy(vals.at[c, s], vals_v)

        st = n_pad // 8
        rel_per_core = nt_stage // n_pad
        spr = n_pad // nt_sub
        tpt = nt_sub // 8
        rl = s // spr
        for i in range(tpt // TB):
            g0 = (s % spr) * tpt + i * TB
            pltpu.sync_copy(
                table.at[pl.ds((c * rel_per_core + rl) * st + g0, TB)], buf8)
            hs = [pltpu.async_copy(
                      buf8.at[pl.ds(0, TB), pl.ds(k * LW, LW)],
                      tab_sh.at[pl.ds(rl * n_pad + k * st + g0, TB)], sm0)
                  for k in range(8)]
            for h in hs:
                h.wait()

        def zrow(i, carry):
            rb0[i, :] = jnp.zeros((LW,), jnp.float32)
            return carry
        lax.fori_loop(0, rc, zrow, 0)
        for t in range(n_rc):
            pltpu.sync_copy(rb0,
                            acc.at[pl.ds(s * rows_per_sub + t * rc, rc)])
        plsc.subcore_barrier()

        slots = ((rb0, gidx_c, src_c, ss0), (rb1, gidx_c2, src_c2, ss1))

        def chunk_half(q, j, rb, gc, sc, ss):
            @pl.when(q > 0)
            def _():
                pltpu.make_async_copy(rb, acc.at[sc.at[0]], ss).wait()
            for g in range(CHUNK // LW):
                v = combo_v[j, pl.ds(g * LW, LW)]
                gc[0, pl.ds(g * LW, LW)] = lax.shift_right_logical(v, 14)
                sc[0, pl.ds(g * LW, LW)] = lax.bitwise_and(v, 16383)
            pltpu.async_copy(tab_sh.at[gc.at[0]], rb, sm0).wait()
            for g in range(CHUNK // LW):
                v16 = vals_v[j, pl.ds(g * LW, LW)]
                for k in range(LW):
                    r = g * LW + k
                    bc = jnp.full((LW,), v16[k], jnp.float32)
                    rb[r, :] = rb[r, :] * bc
            pltpu.async_copy(rb, acc.at[sc.at[0]], ss, add=True)

        def chunk_body(q, carry):
            for b, (rb, gc, sc, ss) in enumerate(slots):
                chunk_half(q, 2 * q + b, rb, gc, sc, ss)
            return carry
        lax.fori_loop(0, nch // 2, chunk_body, 0)
        for rb, gc, sc, ss in slots:
            pltpu.make_async_copy(rb, acc.at[sc.at[0]], ss).wait()
        plsc.subcore_barrier()

        for t in range(n_rc):
            base = s * rows_per_sub + t * rc
            pltpu.sync_copy(acc.at[pl.ds(base, rc)], rb0)
            pltpu.sync_copy(rb0, out.at[c, pl.ds(base, rc)])

    return edge_pass


def kernel(features, W1, W2, bias1, bias2, ln1_g, ln1_b, ln2_g, ln2_b,
           rows, cols, vals):
    N, EMB = features.shape
    RP, _, HID = W1.shape
    NCLS = W2.shape[2]
    E = rows.shape[0]
    R = (RP - 1) // 2
    T = (E - N) // 2

    n_pad = -(-N // (NS * 128)) * NS * 128
    rows32 = rows.astype(jnp.int32)
    cols32 = cols.astype(jnp.int32)
    src = rows32 % N
    gidx = (rows32 - src) // N * n_pad + cols32

    n2r = 2 * R * n_pad
    nt_stage = R * n_pad
    rebase = n2r - nt_stage
    nch = -(-(-(-T // (NS * CHUNK))) // 2) * 2
    epc = NS * nch * CHUNK

    def part(a0, a1):
        a = jnp.concatenate([
            jnp.pad(a0, (0, epc - T)), jnp.pad(a1, (0, epc - T))])
        return a.reshape(NC, NS, nch, CHUNK)

    combo = gidx * 16384 + src
    combo4 = part(combo[:T], combo[T:2 * T] - rebase * 16384)
    vals32 = vals.astype(jnp.float32)
    vals4 = part(vals32[:T], vals32[T:2 * T])

    edge_pass = _make_edge_pass(n_pad, n2r, nt_stage, nch)

    y8, ys = _tc_matmul1(features.astype(jnp.float32), W1, n_pad)
    acc1 = edge_pass(y8, combo4, vals4)

    W2p = jnp.pad(W2, ((0, 0), (0, 0), (0, LW - NCLS)))
    z8, zs = _tc_norm_matmul2(acc1, ys, bias1.reshape(1, HID),
                              ln1_g.reshape(1, HID), ln1_b.reshape(1, HID),
                              W2p, N, n_pad)
    acc2 = edge_pass(z8, combo4, vals4)

    return _tc_final_norm(acc2, zs, bias2.reshape(1, NCLS),
                          ln2_g.reshape(1, NCLS), ln2_b.reshape(1, NCLS),
                          N, NCLS)


# speedup vs baseline: 1.0858x; 1.0007x over previous
import functools

import jax
import jax.numpy as jnp
from jax import lax
from jax.experimental import pallas as pl
from jax.experimental.pallas import tpu as pltpu
from jax.experimental.pallas import tpu_sc as plsc

NC = 2
NS = 16
LW = 16
CHUNK = 128
TB = 64


def _tc_matmul1(x, W1, n_pad):
    RP, EMB, HID = W1.shape
    N = x.shape[0]

    def body(x_ref, w_ref, y8_ref, ys_ref):
        st = n_pad // 8
        for r in range(RP):
            m = jnp.dot(x_ref[...], w_ref[r],
                        preferred_element_type=jnp.float32)
            mp = jnp.pad(m, ((0, n_pad - N), (0, 0)))
            y8_ref[pl.ds(r * st, st), :] = jnp.concatenate(
                [mp[k * st:(k + 1) * st] for k in range(8)], axis=1)
            if r == RP - 1:
                ys_ref[...] = m

    return pl.pallas_call(
        body,
        in_specs=[
            pl.BlockSpec((N, EMB), lambda: (0, 0)),
            pl.BlockSpec((RP, EMB, HID), lambda: (0, 0, 0)),
        ],
        out_specs=[
            pl.BlockSpec((RP * n_pad // 8, 8 * HID), lambda: (0, 0)),
            pl.BlockSpec((N, HID), lambda: (0, 0)),
        ],
        out_shape=[
            jax.ShapeDtypeStruct((RP * n_pad // 8, 8 * HID), jnp.float32),
            jax.ShapeDtypeStruct((N, HID), jnp.float32),
        ],
        compiler_params=pltpu.CompilerParams(
            vmem_limit_bytes=100 * 1024 * 1024),
    )(x, W1)


def _tc_norm_matmul2(acc, yself, b1, g1, bb1, W2p, n, n_pad):
    RP, HID, CP = W2p.shape
    NP = acc.shape[1]

    def body(a_ref, y_ref, b_ref, g_ref, bb_ref, w_ref, z8_ref, zs_ref,
             h_ref):
        @pl.when(pl.program_id(0) == 0)
        def _():
            a = (a_ref[0] + a_ref[1])[:n] + y_ref[...] + b_ref[0]
            mu = jnp.mean(a, axis=-1, keepdims=True)
            var = jnp.mean((a - mu) ** 2, axis=-1, keepdims=True)
            h = (a - mu) * lax.rsqrt(var + 1e-5) * g_ref[0] + bb_ref[0]
            h_ref[...] = jnp.maximum(h, 0.0)

        zr = jnp.dot(h_ref[...], w_ref[0], preferred_element_type=jnp.float32)
        zp = jnp.pad(zr, ((0, n_pad - n), (0, 0)))
        st = n_pad // 8
        z8_ref[...] = jnp.concatenate(
            [zp[k * st:(k + 1) * st] for k in range(8)], axis=1)
        zs_ref[...] = zr

    return pl.pallas_call(
        body,
        grid=(RP,),
        in_specs=[
            pl.BlockSpec((2, NP, HID), lambda r: (0, 0, 0)),
            pl.BlockSpec((n, HID), lambda r: (0, 0)),
            pl.BlockSpec((1, HID), lambda r: (0, 0)),
            pl.BlockSpec((1, HID), lambda r: (0, 0)),
            pl.BlockSpec((1, HID), lambda r: (0, 0)),
            pl.BlockSpec((1, HID, CP), lambda r: (r, 0, 0)),
        ],
        out_specs=[
            pl.BlockSpec((n_pad // 8, 8 * CP), lambda r: (r, 0)),
            pl.BlockSpec((n, CP), lambda r: (0, 0)),
        ],
        out_shape=[
            jax.ShapeDtypeStruct((RP * n_pad // 8, 8 * CP), jnp.float32),
            jax.ShapeDtypeStruct((n, CP), jnp.float32),
        ],
        scratch_shapes=[pltpu.VMEM((n, HID), jnp.float32)],
    )(acc, yself, b1, g1, bb1, W2p)


def _tc_final_norm(acc, zself, b2, g2, bb2, n, ncls):
    NP, CP = acc.shape[1], acc.shape[2]

    def body(a_ref, z_ref, b_ref, g_ref, bb_ref, o_ref):
        a = ((a_ref[0] + a_ref[1])[:n] + z_ref[...])[:, :ncls] + b_ref[0]
        mu = jnp.mean(a, axis=-1, keepdims=True)
        var = jnp.mean((a - mu) ** 2, axis=-1, keepdims=True)
        o_ref[...] = (a - mu) * lax.rsqrt(var + 1e-5) * g_ref[0] + bb_ref[0]

    return pl.pallas_call(
        body,
        in_specs=[
            pl.BlockSpec((2, NP, CP), lambda: (0, 0, 0)),
            pl.BlockSpec((n, CP), lambda: (0, 0)),
            pl.BlockSpec((1, ncls), lambda: (0, 0)),
            pl.BlockSpec((1, ncls), lambda: (0, 0)),
            pl.BlockSpec((1, ncls), lambda: (0, 0)),
        ],
        out_specs=pl.BlockSpec((n, ncls), lambda: (0, 0)),
        out_shape=jax.ShapeDtypeStruct((n, ncls), jnp.float32),
    )(acc, zself, b2, g2, bb2)


def _make_edge_pass(n_pad, n_table, nt_stage, nch):
    rows_per_sub = n_pad // NS
    rc = 128
    n_rc = rows_per_sub // rc
    nt_sub = nt_stage // NS
    mesh = plsc.VectorSubcoreMesh(core_axis_name="c", subcore_axis_name="s")

    @functools.partial(
        pl.kernel,
        mesh=mesh,
        compiler_params=pltpu.CompilerParams(use_tc_tiling_on_sc=False),
        out_type=jax.ShapeDtypeStruct((NC, n_pad, LW), jnp.float32),
        scratch_types=[
            pltpu.VMEM((nch, CHUNK), jnp.int32),
            pltpu.VMEM((nch, CHUNK), jnp.float32),
            pltpu.VMEM((1, CHUNK), jnp.int32),
            pltpu.VMEM((1, CHUNK), jnp.int32),
            pltpu.VMEM((1, CHUNK), jnp.int32),
            pltpu.VMEM((1, CHUNK), jnp.int32),
            pltpu.VMEM((CHUNK, LW), jnp.float32),
            pltpu.VMEM((CHUNK, LW), jnp.float32),
            pltpu.VMEM((TB, 8 * LW), jnp.float32),
            pltpu.VMEM_SHARED((nt_stage, LW), jnp.float32),
            pltpu.VMEM_SHARED((n_pad, LW), jnp.float32),
            pltpu.SemaphoreType.DMA,
            pltpu.SemaphoreType.DMA,
            pltpu.SemaphoreType.DMA,
        ],
    )
    def edge_pass(table, combo, vals, out,
                  combo_v, vals_v, gidx_c, src_c, gidx_c2, src_c2, rb0, rb1,
                  buf8, tab_sh, acc, sm0, ss0, ss1):
        c = lax.axis_index("c")
        s = lax.axis_index("s")

        pltpu.sync_copy(combo.at[c, s], combo_v)
        pltpu.sync_copy(vals.at[c, s], vals_v)

        st = n_pad // 8
        rel_per_core = nt_stage // n_pad
        spr = n_pad // nt_sub
        tpt = nt_sub // 8
        rl = s // spr
        for i in range(tpt // TB):
            g0 = (s % spr) * tpt + i * TB
            pltpu.sync_copy(
                table.at[pl.ds((c * rel_per_core + rl) * st + g0, TB)], buf8)
            hs = [pltpu.async_copy(
                      buf8.at[pl.ds(0, TB), pl.ds(k * LW, LW)],
                      tab_sh.at[pl.ds(rl * n_pad + k * st + g0, TB)], sm0)
                  for k in range(8)]
            for h in hs:
                h.wait()

        def zrow(i, carry):
            rb0[i, :] = jnp.zeros((LW,), jnp.float32)
            return carry
        lax.fori_loop(0, rc, zrow, 0)
        for t in range(n_rc):
            pltpu.sync_copy(rb0,
                            acc.at[pl.ds(s * rows_per_sub + t * rc, rc)])
        plsc.subcore_barrier()

        slots = ((rb0, gidx_c, src_c, ss0), (rb1, gidx_c2, src_c2, ss1))

        def chunk_half(q, j, rb, gc, sc, ss):
            @pl.when(q > 0)
            def _():
                pltpu.make_async_copy(rb, acc.at[sc.at[0]], ss).wait()
            for g in range(CHUNK // LW):
                v = combo_v[j, pl.ds(g * LW, LW)]
                gc[0, pl.ds(g * LW, LW)] = lax.shift_right_logical(v, 14)
                sc[0, pl.ds(g * LW, LW)] = lax.bitwise_and(v, 16383)
            pltpu.async_copy(tab_sh.at[gc.at[0]], rb, sm0).wait()
            for g in range(CHUNK // LW):
                v16 = vals_v[j, pl.ds(g * LW, LW)]
                for k in range(LW):
                    r = g * LW + k
                    bc = jnp.full((LW,), v16[k], jnp.float32)
                    rb[r, :] = rb[r, :] * bc
            pltpu.async_copy(rb, acc.at[sc.at[0]], ss, add=True)

        def chunk_body(q, carry):
            for b, (rb, gc, sc, ss) in enumerate(slots):
                chunk_half(q, 2 * q + b, rb, gc, sc, ss)
            return carry
        lax.fori_loop(0, nch // 2, chunk_body, 0)
        for rb, gc, sc, ss in slots:
            pltpu.make_async_copy(rb, acc.at[sc.at[0]], ss).wait()
        plsc.subcore_barrier()

        for t in range(n_rc):
            base = s * rows_per_sub + t * rc
            pltpu.sync_copy(acc.at[pl.ds(base, rc)], rb0)
            pltpu.sync_copy(rb0, out.at[c, pl.ds(base, rc)])

    return edge_pass


def kernel(features, W1, W2, bias1, bias2, ln1_g, ln1_b, ln2_g, ln2_b,
           rows, cols, vals):
    N, EMB = features.shape
    RP, _, HID = W1.shape
    NCLS = W2.shape[2]
    E = rows.shape[0]
    R = (RP - 1) // 2
    T = (E - N) // 2

    n_pad = -(-N // (NS * 128)) * NS * 128
    rows32 = rows.astype(jnp.int32)
    cols32 = cols.astype(jnp.int32)
    src = rows32 % N
    gidx = (rows32 - src) // N * n_pad + cols32

    n2r = 2 * R * n_pad
    nt_stage = R * n_pad
    rebase = n2r - nt_stage
    nch = -(-(-(-T // (NS * CHUNK))) // 2) * 2
    epc = NS * nch * CHUNK

    def part(a0, a1):
        a = jnp.concatenate([
            jnp.pad(a0, (0, epc - T)), jnp.pad(a1, (0, epc - T))])
        return a.reshape(NC, NS, nch, CHUNK)

    combo = gidx * 16384 + src
    combo4 = part(combo[:T], combo[T:2 * T] - rebase * 16384)
    vals32 = vals.astype(jnp.float32)
    vals4 = part(vals32[:T], vals32[T:2 * T])

    edge_pass = _make_edge_pass(n_pad, n2r, nt_stage, nch)

    y8, ys = _tc_matmul1(features.astype(jnp.float32), W1, n_pad)
    acc1 = edge_pass(y8, combo4, vals4)

    W2p = jnp.pad(W2, ((0, 0), (0, 0), (0, LW - NCLS)))
    z8, zs = _tc_norm_matmul2(acc1, ys, bias1.reshape(1, HID),
                              ln1_g.reshape(1, HID), ln1_b.reshape(1, HID),
                              W2p, N, n_pad)
    acc2 = edge_pass(z8, combo4, vals4)

    return _tc_final_norm(acc2, zs, bias2.reshape(1, NCLS),
                          ln2_g.reshape(1, NCLS), ln2_b.reshape(1, NCLS),
                          N, NCLS)
